# baseline (speedup 1.0000x reference)
import functools

import jax
import jax.numpy as jnp
from jax import lax
from jax.experimental import pallas as pl
from jax.experimental.pallas import tpu as pltpu
from jax.experimental.pallas import tpu_sc as plsc

_VOCAB = 1000000
_REPR = 64
_BATCH = 16384

_NC = 2
_NS = 16
_L = 16
_NW = _NC * _NS
_BPW = _BATCH // _NW
_GCH = 128
_NG = _BPW // _GCH

_mesh = plsc.VectorSubcoreMesh(core_axis_name="c", subcore_axis_name="s")


@functools.partial(
    pl.kernel,
    mesh=_mesh,
    compiler_params=pltpu.CompilerParams(use_tc_tiling_on_sc=False),
    out_type=jax.ShapeDtypeStruct((_NW, _L), jnp.float32),
    scratch_types=[
        pltpu.VMEM((_BPW,), jnp.int32),
        pltpu.VMEM((_BPW,), jnp.int32),
        pltpu.VMEM((_BPW, _REPR), jnp.float32),
        pltpu.VMEM((_BPW, _REPR), jnp.float32),
        pltpu.VMEM((_BPW, _REPR), jnp.float32),
        pltpu.VMEM((_L,), jnp.float32),
        pltpu.SemaphoreType.DMA,
    ],
)
def _mse_partials(rep_hbm, ia_hbm, ib_hbm, emb_hbm, out_hbm,
                  ia_v, ib_v, ea_v, eb_v, rep_v, acc_v, sem):
    wid = lax.axis_index("s") * _NC + lax.axis_index("c")
    base = wid * _BPW

    pltpu.sync_copy(ia_hbm.at[pl.ds(base, _BPW)], ia_v)
    pltpu.sync_copy(ib_hbm.at[pl.ds(base, _BPW)], ib_v)

    copies = []
    for g in range(_NG):
        sl = pl.ds(g * _GCH, _GCH)
        copies.append(pltpu.async_copy(emb_hbm.at[ia_v.at[sl]], ea_v.at[sl], sem))
        copies.append(pltpu.async_copy(emb_hbm.at[ib_v.at[sl]], eb_v.at[sl], sem))
    copies.append(pltpu.async_copy(rep_hbm.at[pl.ds(base, _BPW)], rep_v, sem))
    for c in copies:
        c.wait()

    def body(i, acc):
        for j in range(_REPR // _L):
            sl = pl.ds(j * _L, _L)
            d = ea_v[i, sl] + eb_v[i, sl] - rep_v[i, sl]
            acc = acc + d * d
        return acc

    acc = lax.fori_loop(0, _BPW, body, jnp.zeros((_L,), jnp.float32))
    acc_v[...] = acc
    pltpu.sync_copy(acc_v, out_hbm.at[wid])


def kernel(rep, idx_a, idx_b, emb):
    partials = _mse_partials(rep, idx_a.astype(jnp.int32),
                             idx_b.astype(jnp.int32), emb)
    return jnp.sum(partials) / jnp.float32(_BATCH * _REPR)
